# baseline (speedup 1.0000x reference)
import functools

import jax
import jax.numpy as jnp
from jax import lax
from jax.experimental import pallas as pl
from jax.experimental.pallas import tpu as pltpu
from jax.experimental.pallas import tpu_sc as plsc

VOCAB = 100000
D = 64
B = 4096
H = 50
C = 6
NHEAD = 3
EPS = 1e-09

NC = 2
NS = 16
NW = NC * NS
BAGS = NHEAD * B
BPW = BAGS // NW
CB = 16
RPC = CB * H
NCHUNK = BPW // CB
CPH = B // CB
SUB_ROWS = 80
NSUB = RPC // SUB_ROWS
LANES = 16
DSUB = D // LANES


def _bagsum_body(h0_hbm, h1_hbm, h2_hbm, table_hbm, out_hbm,
                 idx_v, rows_v, out_v, sem0, sem1):
    wid = lax.axis_index("s") * NC + lax.axis_index("c")
    gc0 = wid * NCHUNK
    hrefs = (h0_hbm, h1_hbm, h2_hbm)
    sems = (sem0, sem1)

    def issue(slot, gc):
        boff = (gc % CPH) * CB
        for hsel in range(NHEAD):
            @pl.when(gc // CPH == hsel)
            def _():
                pltpu.sync_copy(hrefs[hsel].at[pl.ds(boff, CB), :],
                                idx_v.at[slot])
        for b in range(CB):
            pltpu.async_copy(table_hbm.at[idx_v.at[slot, b]],
                             rows_v.at[slot, b], sems[slot])

    def compute(slot, gc):
        for b in range(CB):
            pltpu.make_async_copy(
                table_hbm.at[idx_v.at[slot, b]],
                rows_v.at[slot, b], sems[slot]).wait()

        def bag_body(bag, carry2):
            accs = [jnp.zeros((LANES,), jnp.float32) for _ in range(DSUB)]
            for hh in range(H):
                for k in range(DSUB):
                    accs[k] = accs[k] + rows_v[slot, bag, hh,
                                               pl.ds(k * LANES, LANES)]
            for k in range(DSUB):
                out_v[bag, pl.ds(k * LANES, LANES)] = accs[k]
            return carry2

        lax.fori_loop(0, CB, bag_body, 0)
        pltpu.sync_copy(out_v, out_hbm.at[pl.ds(gc * CB, CB)])

    issue(0, gc0)

    def pair_body(p, carry):
        ga = gc0 + 2 * p
        issue(1, ga + 1)
        compute(0, ga)

        @pl.when(p < NCHUNK // 2 - 1)
        def _():
            issue(0, ga + 2)

        compute(1, ga + 1)
        return carry

    lax.fori_loop(0, NCHUNK // 2, pair_body, 0)


@functools.lru_cache(maxsize=1)
def _make_bagsum():
    return pl.kernel(
        _bagsum_body,
        mesh=plsc.VectorSubcoreMesh(core_axis_name="c", subcore_axis_name="s"),
        out_type=jax.ShapeDtypeStruct((BAGS, D), jnp.float32),
        scratch_types=[
            pltpu.VMEM((2, CB, H), jnp.int32),
            pltpu.VMEM((2, CB, H, D), jnp.float32),
            pltpu.VMEM((CB, D), jnp.float32),
            pltpu.SemaphoreType.DMA,
            pltpu.SemaphoreType.DMA,
        ],
        compiler_params=pltpu.CompilerParams(use_tc_tiling_on_sc=False),
    )


def _tail_body(sums_ref, l0_ref, l1_ref, l2_ref,
               w0_ref, w1_ref, w2_ref, b0_ref, b1_ref, b2_ref,
               loss_ref, f1_ref, acc_ref):
    labs = (l0_ref, l1_ref, l2_ref)
    ws = (w0_ref, w1_ref, w2_ref)
    bs = (b0_ref, b1_ref, b2_ref)

    loss_sum = jnp.float32(0.0)
    correct = jnp.float32(0.0)
    pos_tp = jnp.float32(0.0); pos_fp = jnp.float32(0.0); pos_fn = jnp.float32(0.0)
    neg_tp = jnp.float32(0.0); neg_fp = jnp.float32(0.0); neg_fn = jnp.float32(0.0)

    for i in range(NHEAD):
        s = sums_ref[pl.ds(i * B, B), :]
        sq = jnp.sum(s * s, axis=1, keepdims=True)
        normed = s * lax.rsqrt(jnp.maximum(sq, 1e-12))
        logits = jnp.dot(normed, ws[i][...],
                         preferred_element_type=jnp.float32) + bs[i][...]
        pred = jnp.clip(jax.nn.sigmoid(logits), EPS, 1.0 - EPS)
        lab = labs[i][...]
        loss = -lab * jnp.log(pred) - (1.0 - lab) * jnp.log(1.0 - pred)
        loss_sum = loss_sum + jnp.sum(loss) / jnp.float32(B)

        pred_label = pred > 0.5
        bool_label = lab == 1.0
        correct = correct + jnp.sum((pred_label == bool_label).astype(jnp.float32))
        pos_tp = pos_tp + jnp.sum(jnp.logical_and(bool_label, pred_label).astype(jnp.float32))
        pos_fp = pos_fp + jnp.sum(jnp.logical_and(jnp.logical_not(bool_label), pred_label).astype(jnp.float32))
        pos_fn = pos_fn + jnp.sum(jnp.logical_and(bool_label, jnp.logical_not(pred_label)).astype(jnp.float32))

        pred_label_n = pred < 0.5
        bool_label_n = lab == 0.0
        neg_tp = neg_tp + jnp.sum(jnp.logical_and(bool_label_n, pred_label_n).astype(jnp.float32))
        neg_fp = neg_fp + jnp.sum(jnp.logical_and(jnp.logical_not(bool_label_n), pred_label_n).astype(jnp.float32))
        neg_fn = neg_fn + jnp.sum(jnp.logical_and(bool_label_n, jnp.logical_not(pred_label_n)).astype(jnp.float32))

    accuracy = correct / jnp.float32(B * NHEAD * C)
    pos_recall = pos_tp / jnp.maximum(EPS, pos_tp + pos_fn)
    pos_precision = pos_tp / jnp.maximum(EPS, pos_tp + pos_fp)
    pos_f1 = 2 * pos_recall * pos_precision / jnp.maximum(EPS, pos_recall + pos_precision)
    neg_recall = neg_tp / jnp.maximum(EPS, neg_tp + neg_fn)
    neg_precision = neg_tp / jnp.maximum(EPS, neg_tp + neg_fp)
    neg_f1 = 2 * neg_recall * neg_precision / jnp.maximum(EPS, neg_recall + neg_precision)

    loss_ref[0, 0] = loss_sum
    f1_ref[0, 0] = (pos_f1 + neg_f1) / 2.0
    acc_ref[0, 0] = accuracy


def _make_tail(interpret=False):
    return pl.pallas_call(
        _tail_body,
        out_shape=[jax.ShapeDtypeStruct((1, 1), jnp.float32)] * 3,
        in_specs=[pl.BlockSpec(memory_space=pltpu.VMEM)] * 10,
        out_specs=[pl.BlockSpec(memory_space=pltpu.SMEM)] * 3,
        interpret=interpret,
    )


_tail = _make_tail()


def kernel(unique_emb, history_0, history_1, history_2,
           label_0, label_1, label_2,
           W_0, b_0, W_1, b_1, W_2, b_2):
    sums = _make_bagsum()(history_0, history_1, history_2, unique_emb)
    loss, f1, acc = _tail(sums, label_0, label_1, label_2,
                          W_0, W_1, W_2,
                          b_0.reshape(1, C), b_1.reshape(1, C),
                          b_2.reshape(1, C))
    return loss[0, 0], f1[0, 0], acc[0, 0]


# speedup vs baseline: 1.1449x; 1.1449x over previous
import functools

import jax
import jax.numpy as jnp
from jax import lax
from jax.experimental import pallas as pl
from jax.experimental.pallas import tpu as pltpu
from jax.experimental.pallas import tpu_sc as plsc

VOCAB = 100000
D = 64
B = 4096
H = 50
C = 6
NHEAD = 3
EPS = 1e-09

NC = 2
NS = 16
NW = NC * NS
BAGS = NHEAD * B
BPH = B // NW
BPW = NHEAD * BPH
CB = 16
NCHUNK = BPW // CB
CPH = BPH // CB
LANES = 16
DSUB = D // LANES
OD = 128


def _bagsum_body(h0_hbm, h1_hbm, h2_hbm, table_hbm, out_hbm,
                 idx_all, rows_v, out_v, sem0, sem1):
    wid = lax.axis_index("s") * NC + lax.axis_index("c")
    bag0 = wid * BPH
    hrefs = (h0_hbm, h1_hbm, h2_hbm)
    sems = (sem0, sem1)

    for h in range(NHEAD):
        pltpu.sync_copy(hrefs[h].at[pl.ds(bag0, BPH), :],
                        idx_all.at[pl.ds(h * BPH, BPH), :])

    def issue(slot, g):
        for b in range(CB):
            pltpu.async_copy(table_hbm.at[idx_all.at[g * CB + b]],
                             rows_v.at[slot, b], sems[slot])

    def compute(slot, g):
        for b in range(CB):
            pltpu.make_async_copy(
                table_hbm.at[idx_all.at[g * CB + b]],
                rows_v.at[slot, b], sems[slot]).wait()

        def bag_body(bag, carry2):
            accs = [jnp.zeros((LANES,), jnp.float32) for _ in range(DSUB)]
            for hh in range(H):
                for k in range(DSUB):
                    accs[k] = accs[k] + rows_v[slot, bag, hh,
                                               pl.ds(k * LANES, LANES)]
            for k in range(DSUB):
                out_v[bag, pl.ds(k * LANES, LANES)] = accs[k]
            return carry2

        lax.fori_loop(0, CB, bag_body, 0)
        row0 = (g // CPH) * B + bag0 + (g % CPH) * CB
        pltpu.sync_copy(out_v, out_hbm.at[pl.ds(row0, CB), :])

    issue(0, 0)

    def pair_body(p, carry):
        ga = 2 * p
        issue(1, ga + 1)
        compute(0, ga)

        @pl.when(p < NCHUNK // 2 - 1)
        def _():
            issue(0, ga + 2)

        compute(1, ga + 1)
        return carry

    lax.fori_loop(0, NCHUNK // 2, pair_body, 0)


@functools.lru_cache(maxsize=1)
def _make_bagsum():
    return pl.kernel(
        _bagsum_body,
        mesh=plsc.VectorSubcoreMesh(core_axis_name="c", subcore_axis_name="s"),
        out_type=jax.ShapeDtypeStruct((BAGS, OD), jnp.float32),
        scratch_types=[
            pltpu.VMEM((BPW, H), jnp.int32),
            pltpu.VMEM((2, CB, H, D), jnp.float32),
            pltpu.VMEM((CB, OD), jnp.float32),
            pltpu.SemaphoreType.DMA,
            pltpu.SemaphoreType.DMA,
        ],
        compiler_params=pltpu.CompilerParams(use_tc_tiling_on_sc=False),
    )


def _tail_body(sums_ref, l0_ref, l1_ref, l2_ref,
               w0_ref, w1_ref, w2_ref, b0_ref, b1_ref, b2_ref,
               loss_ref, f1_ref, acc_ref):
    labs = (l0_ref, l1_ref, l2_ref)
    ws = (w0_ref, w1_ref, w2_ref)
    bs = (b0_ref, b1_ref, b2_ref)
    n_all = jnp.float32(B * C)

    loss_sum = jnp.float32(0.0)
    correct = jnp.float32(0.0)
    pos_tp = jnp.float32(0.0); pos_fp = jnp.float32(0.0); pos_fn = jnp.float32(0.0)
    neg_tp = jnp.float32(0.0); neg_fp = jnp.float32(0.0); neg_fn = jnp.float32(0.0)

    for i in range(NHEAD):
        s = sums_ref[pl.ds(i * B, B), pl.ds(0, D)]
        sq = jnp.sum(s * s, axis=1, keepdims=True)
        normed = s * lax.rsqrt(jnp.maximum(sq, 1e-12))
        logits = jnp.dot(normed, ws[i][...],
                         preferred_element_type=jnp.float32) + bs[i][...]
        pred = jnp.clip(jax.nn.sigmoid(logits), EPS, 1.0 - EPS)
        lab = labs[i][...]
        q = lab * pred + (1.0 - lab) * (1.0 - pred)
        loss_sum = loss_sum - jnp.sum(jnp.log(q)) / jnp.float32(B)

        pl_f = (pred > 0.5).astype(jnp.float32)
        pln_f = (pred < 0.5).astype(jnp.float32)
        s1 = jnp.sum(pl_f * lab)
        s2 = jnp.sum(pl_f)
        s3 = jnp.sum(lab)
        t1 = jnp.sum(pln_f * (1.0 - lab))
        t2 = jnp.sum(pln_f)
        pos_tp = pos_tp + s1
        pos_fp = pos_fp + (s2 - s1)
        pos_fn = pos_fn + (s3 - s1)
        neg_tp = neg_tp + t1
        neg_fp = neg_fp + (t2 - t1)
        neg_fn = neg_fn + (n_all - s3 - t1)
        correct = correct + (n_all - s2 - s3 + 2.0 * s1)

    accuracy = correct / jnp.float32(B * NHEAD * C)
    pos_recall = pos_tp / jnp.maximum(EPS, pos_tp + pos_fn)
    pos_precision = pos_tp / jnp.maximum(EPS, pos_tp + pos_fp)
    pos_f1 = 2 * pos_recall * pos_precision / jnp.maximum(EPS, pos_recall + pos_precision)
    neg_recall = neg_tp / jnp.maximum(EPS, neg_tp + neg_fn)
    neg_precision = neg_tp / jnp.maximum(EPS, neg_tp + neg_fp)
    neg_f1 = 2 * neg_recall * neg_precision / jnp.maximum(EPS, neg_recall + neg_precision)

    loss_ref[0, 0] = loss_sum
    f1_ref[0, 0] = (pos_f1 + neg_f1) / 2.0
    acc_ref[0, 0] = accuracy


def _make_tail(interpret=False):
    return pl.pallas_call(
        _tail_body,
        out_shape=[jax.ShapeDtypeStruct((1, 1), jnp.float32)] * 3,
        in_specs=[pl.BlockSpec(memory_space=pltpu.VMEM)] * 10,
        out_specs=[pl.BlockSpec(memory_space=pltpu.SMEM)] * 3,
        interpret=interpret,
    )


_tail = _make_tail()


def kernel(unique_emb, history_0, history_1, history_2,
           label_0, label_1, label_2,
           W_0, b_0, W_1, b_1, W_2, b_2):
    sums = _make_bagsum()(history_0, history_1, history_2, unique_emb)
    loss, f1, acc = _tail(sums, label_0, label_1, label_2,
                          W_0, W_1, W_2,
                          b_0.reshape(1, C), b_1.reshape(1, C),
                          b_2.reshape(1, C))
    return loss[0, 0], f1[0, 0], acc[0, 0]
